# baseline (speedup 1.0000x reference)
import functools

import jax
import jax.numpy as jnp
from jax import lax
from jax.experimental import pallas as pl
from jax.experimental.pallas import tpu as pltpu
from jax.experimental.pallas import tpu_sc as plsc

I = 1024
S = 1024
O = 1024
C = 4
RG = (3 * S) // C

NW = 32
RPW = O // NW
CH = 8
NCH = RPW // CH

_sc_mesh = plsc.VectorSubcoreMesh(core_axis_name="c", subcore_axis_name="s")


@functools.partial(
    pl.kernel,
    out_type=jax.ShapeDtypeStruct((O,), jnp.float32),
    mesh=_sc_mesh,
    compiler_params=pltpu.CompilerParams(needs_layout_passes=False),
    scratch_types=[
        pltpu.VMEM((3 * S,), jnp.float32),
        pltpu.VMEM((2, CH, 3073), jnp.float32),
        pltpu.VMEM((RPW,), jnp.float32),
        pltpu.SemaphoreType.DMA,
        pltpu.SemaphoreType.DMA,
    ],
)
def _sc_logits(wout_hbm, f_hbm, h_hbm, out_hbm, x_v, w_v, out_v, sem0, sem1):
    wid = lax.axis_index("s") * 2 + lax.axis_index("c")
    base = wid * RPW
    sems = (sem0, sem1)

    copies = [
        pltpu.async_copy(wout_hbm.at[pl.ds(base + b * CH, CH)], w_v.at[b],
                         sems[b])
        for b in range(2)
    ]
    pltpu.sync_copy(f_hbm, x_v.at[pl.ds(0, S)])
    pltpu.sync_copy(h_hbm, x_v.at[pl.ds(S, S)])
    pltpu.sync_copy(h_hbm, x_v.at[pl.ds(2 * S, S)])

    lane = lax.broadcasted_iota(jnp.int32, (16,), 0)
    z = jnp.zeros((16,), jnp.float32)
    vecs = [z, z]
    for ch in range(NCH):
        b = ch % 2
        copies[b].wait()
        for r0 in range(0, CH, 4):
            def body(i, accs, _b=b, _r0=r0):
                off = pl.multiple_of(i * 16, 16)
                xv = x_v[pl.ds(off, 16)]
                return tuple(
                    accs[k] + w_v[_b, _r0 + k, pl.ds(off, 16)] * xv
                    for k in range(4)
                )
            a = lax.fori_loop(0, (3 * S) // 16, body, (z, z, z, z), unroll=8)
            for k in range(4):
                row = ch * CH + r0 + k
                vecs[row // 16] = jnp.where(lane == (row % 16),
                                            jnp.sum(a[k]), vecs[row // 16])
        nxt = ch + 2
        if nxt < NCH:
            copies[b] = pltpu.async_copy(
                wout_hbm.at[pl.ds(base + nxt * CH, CH)], w_v.at[b], sems[b])
    out_v[pl.ds(0, 16)] = vecs[0]
    out_v[pl.ds(16, 16)] = vecs[1]
    pltpu.sync_copy(out_v, out_hbm.at[pl.ds(base, RPW)])


def _tc_gru(f_ref, h_ref, wih_hbm, whh_hbm, bih_ref, bhh_ref, out_h_ref,
            wih_v, whh_v, gi_ref, gh_ref, sems):
    for c in range(C):
        pltpu.make_async_copy(
            wih_hbm.at[pl.ds(c * RG, RG), :], wih_v.at[pl.ds(c * RG, RG), :],
            sems.at[2 * c]).start()
        pltpu.make_async_copy(
            whh_hbm.at[pl.ds(c * RG, RG), :], whh_v.at[pl.ds(c * RG, RG), :],
            sems.at[2 * c + 1]).start()

    f_row = f_ref[...]
    h_row = h_ref[...]

    for c in range(C):
        pltpu.make_async_copy(
            wih_hbm.at[pl.ds(c * RG, RG), :], wih_v.at[pl.ds(c * RG, RG), :],
            sems.at[2 * c]).wait()
        gi = jnp.sum(wih_v[pl.ds(c * RG, RG), :] * f_row, axis=1,
                     keepdims=True)
        gi_ref[0, pl.ds(c * RG, RG)] = jnp.transpose(gi)[0]

        pltpu.make_async_copy(
            whh_hbm.at[pl.ds(c * RG, RG), :], whh_v.at[pl.ds(c * RG, RG), :],
            sems.at[2 * c + 1]).wait()
        gh = jnp.sum(whh_v[pl.ds(c * RG, RG), :] * h_row, axis=1,
                     keepdims=True)
        gh_ref[0, pl.ds(c * RG, RG)] = jnp.transpose(gh)[0]

    gi_full = gi_ref[...] + bih_ref[...]
    gh_full = gh_ref[...] + bhh_ref[...]
    r = jax.nn.sigmoid(gi_full[:, :S] + gh_full[:, :S])
    z = jax.nn.sigmoid(gi_full[:, S:2 * S] + gh_full[:, S:2 * S])
    n = jnp.tanh(gi_full[:, 2 * S:] + r * gh_full[:, 2 * S:])
    out_h_ref[...] = (1.0 - z) * n + z * h_row


def _tc_logsoftmax(lg_ref, bout_ref, out_ref):
    logits = lg_ref[...] + bout_ref[...]
    m = jnp.max(logits)
    lse = jnp.log(jnp.sum(jnp.exp(logits - m))) + m
    out_ref[...] = logits - lse


def kernel(feature, time, initial_h, W_ih, W_hh, b_ih, b_hh, W_out, b_out):
    del time
    f_row = feature.reshape(1, I)
    h_row = initial_h.reshape(1, S)
    bih_row = b_ih.reshape(1, 3 * S)
    bhh_row = b_hh.reshape(1, 3 * S)
    bout_row = b_out.reshape(1, O)

    logits_raw = _sc_logits(W_out, feature, initial_h)

    vmem = pl.BlockSpec(memory_space=pltpu.MemorySpace.VMEM)
    hbm = pl.BlockSpec(memory_space=pltpu.MemorySpace.HBM)

    out_h = pl.pallas_call(
        _tc_gru,
        in_specs=[vmem, vmem, hbm, hbm, vmem, vmem],
        out_specs=vmem,
        out_shape=jax.ShapeDtypeStruct((1, S), jnp.float32),
        scratch_shapes=[
            pltpu.MemorySpace.VMEM((3 * S, I), jnp.float32),
            pltpu.MemorySpace.VMEM((3 * S, S), jnp.float32),
            pltpu.MemorySpace.VMEM((1, 3 * S), jnp.float32),
            pltpu.MemorySpace.VMEM((1, 3 * S), jnp.float32),
            pltpu.SemaphoreType.DMA((2 * C,)),
        ],
    )(f_row, h_row, W_ih, W_hh, bih_row, bhh_row)

    out_logp = pl.pallas_call(
        _tc_logsoftmax,
        in_specs=[vmem, vmem],
        out_specs=vmem,
        out_shape=jax.ShapeDtypeStruct((1, O), jnp.float32),
    )(logits_raw.reshape(1, O), bout_row)

    return (out_logp, out_h.reshape(1, 1, S))
